# baseline (speedup 1.0000x reference)
import functools
import math

import jax
import jax.numpy as jnp
from jax import lax
from jax.experimental import pallas as pl
from jax.experimental.pallas import tpu as pltpu
from jax.experimental.pallas import tpu_sc as plsc

_SIZE = 32000
_PAD_IDX = 0
_SMOOTHING = 0.1
_CONFIDENCE = 1.0 - _SMOOTHING
_S = _SMOOTHING / (_SIZE - 2)
_C0 = _SMOOTHING * math.log(_S) + _CONFIDENCE * math.log(_CONFIDENCE)

_BR = 64
_NC, _NS = 2, 16
_NW = _NC * _NS
_N_TOK = 8192
_BPW = _N_TOK // _NW


def _rowsum_kernel(t_ref, x_ref, o_ref):
    ri = pl.program_id(0)

    @pl.when(ri == 0)
    def _init():
        o_ref[...] = jnp.zeros_like(o_ref)

    x = x_ref[...]
    maskf = (t_ref[...] != _PAD_IDX).astype(jnp.float32)
    rs = jnp.sum(x, axis=1, keepdims=True)
    part = jnp.sum(maskf * (jnp.float32(_S) * x[:, 0:1] - jnp.float32(_S) * rs))
    o_ref[...] += part.reshape(1, 1)


def _sc_body(x_hbm, t_hbm, o_hbm, t_v, idx_v, val_v, acc_v, sem):
    c = lax.axis_index("c")
    s = lax.axis_index("s")
    wid = s * _NC + c
    base = wid * _BPW
    pltpu.sync_copy(t_hbm.at[pl.ds(base, _BPW)], t_v)

    def mk_idx(j, carry):
        t = t_v[pl.ds(j * 16, 16)]
        rows = base + j * 16 + lax.iota(jnp.int32, 16)
        idx_v[pl.ds(j * 16, 16)] = rows * _SIZE + t
        return carry

    lax.fori_loop(0, _BPW // 16, mk_idx, 0)

    for g in range(_BPW // 128):
        pltpu.async_copy(
            x_hbm.at[idx_v.at[pl.ds(g * 128, 128)]],
            val_v.at[pl.ds(g * 128, 128)], sem).wait()

    def accum(j, acc):
        t = t_v[pl.ds(j * 16, 16)]
        v = val_v[pl.ds(j * 16, 16)]
        contrib = jnp.float32(_S - _CONFIDENCE) * v + jnp.float32(_C0)
        return acc + jnp.where(t != _PAD_IDX, contrib, jnp.float32(0.0))

    acc_v[...] = lax.fori_loop(0, _BPW // 16, accum, jnp.zeros((16,), jnp.float32))
    pltpu.sync_copy(acc_v, o_hbm.at[wid])


@functools.partial(
    pl.kernel,
    mesh=plsc.VectorSubcoreMesh(core_axis_name="c", subcore_axis_name="s"),
    out_type=jax.ShapeDtypeStruct((_NW, 16), jnp.float32),
    scratch_types=[
        pltpu.VMEM((_BPW,), jnp.int32),
        pltpu.VMEM((_BPW,), jnp.int32),
        pltpu.VMEM((_BPW,), jnp.float32),
        pltpu.VMEM((16,), jnp.float32),
        pltpu.SemaphoreType.DMA,
    ],
)
def _sc_gather(x_hbm, t_hbm, o_hbm, t_v, idx_v, val_v, acc_v, sem):
    _sc_body(x_hbm, t_hbm, o_hbm, t_v, idx_v, val_v, acc_v, sem)


def _combine_kernel(a_ref, b_ref, o_ref):
    o_ref[...] = a_ref[...] + jnp.sum(b_ref[...]).reshape(1, 1)


def kernel(x, target):
    n, v = x.shape
    t2 = target.reshape(n, 1)
    tc_part = pl.pallas_call(
        _rowsum_kernel,
        grid=(n // _BR,),
        in_specs=[
            pl.BlockSpec((_BR, 1), lambda i: (i, 0)),
            pl.BlockSpec((_BR, v), lambda i: (i, 0)),
        ],
        out_specs=pl.BlockSpec((1, 1), lambda i: (0, 0)),
        out_shape=jax.ShapeDtypeStruct((1, 1), jnp.float32),
    )(t2, x)
    sc_parts = _sc_gather(x.reshape(-1), target)
    out = pl.pallas_call(
        _combine_kernel,
        out_shape=jax.ShapeDtypeStruct((1, 1), jnp.float32),
    )(tc_part, sc_parts)
    return out.reshape(())


# speedup vs baseline: 3.2031x; 3.2031x over previous
import functools
import math

import jax
import jax.numpy as jnp
from jax import lax
from jax.experimental import pallas as pl

_SIZE = 32000
_PAD_IDX = 0
_SMOOTHING = 0.1
_CONFIDENCE = 1.0 - _SMOOTHING
_S = _SMOOTHING / (_SIZE - 2)
_C0 = _SMOOTHING * math.log(_S) + _CONFIDENCE * math.log(_CONFIDENCE)

_BR = 64


def _ls_kernel(t_ref, x_ref, o_ref):
    ri = pl.program_id(0)

    @pl.when(ri == 0)
    def _init():
        o_ref[...] = jnp.zeros_like(o_ref)

    x = x_ref[...]
    t = t_ref[...]
    maskf = (t != _PAD_IDX).astype(jnp.float32)
    cols = lax.broadcasted_iota(jnp.int32, x.shape, 1)
    coef = jnp.where(cols == t, jnp.float32(-_CONFIDENCE), jnp.float32(-_S))
    wrow = jnp.sum(x * coef, axis=1, keepdims=True)
    part = jnp.sum(maskf * (wrow + jnp.float32(_S) * x[:, 0:1] + jnp.float32(_C0)))
    o_ref[...] += part.reshape(1, 1)


def kernel(x, target):
    n, v = x.shape
    t2 = target.reshape(n, 1)
    out = pl.pallas_call(
        _ls_kernel,
        grid=(n // _BR,),
        in_specs=[
            pl.BlockSpec((_BR, 1), lambda i: (i, 0)),
            pl.BlockSpec((_BR, v), lambda i: (i, 0)),
        ],
        out_specs=pl.BlockSpec((1, 1), lambda i: (0, 0)),
        out_shape=jax.ShapeDtypeStruct((1, 1), jnp.float32),
    )(t2, x)
    return out.reshape(())
